# Initial kernel scaffold; baseline (speedup 1.0000x reference)
import jax
import jax.numpy as jnp
from jax.experimental import pallas as pl


def kernel(q, local_repr, context):
    raise NotImplementedError("write your pallas kernel here")


# speedup vs baseline: 2.0423x; 2.0423x over previous
import jax
import jax.numpy as jnp
from jax.experimental import pallas as pl
from jax.experimental.pallas import tpu as pltpu

_B = 1024
_Q = 8
_D = 256
_K = _Q * _D
_N = 8192
_BN = 512
_NB = _N // _BN


def _vq_min_kernel(qt_ref, g_ref, ctx_ref, out_ref, cpy_ref, acc_ref):
    n = pl.program_id(0)

    cpy_ref[...] = ctx_ref[...]

    qt = qt_ref[...]
    gb = g_ref[...]

    dots = jnp.dot(gb, qt, preferred_element_type=jnp.float32)
    gnorm = jnp.sum(gb.astype(jnp.float32) ** 2, axis=1,
                    keepdims=True)
    d = gnorm - 2.0 * dots
    m = jnp.min(d, axis=0, keepdims=True)

    @pl.when(n == 0)
    def _init():
        acc_ref[...] = m

    @pl.when(n > 0)
    def _update():
        acc_ref[...] = jnp.minimum(acc_ref[...], m)

    @pl.when(n == _NB - 1)
    def _finish():
        qn = jnp.sum(qt.astype(jnp.float32) ** 2, axis=0,
                     keepdims=True)
        out_ref[...] = (qn + acc_ref[...]) * (1.0 / _K)


def kernel(q, local_repr, context):
    del local_repr
    qt = q.reshape(_B, _K).astype(jnp.bfloat16).T
    g_flat = context.reshape(_N, _K).astype(jnp.bfloat16)
    ctx2d = context.reshape(_N * _Q, _D)

    out1, ctx_out = pl.pallas_call(
        _vq_min_kernel,
        grid=(_NB,),
        in_specs=[
            pl.BlockSpec((_K, _B), lambda n: (0, 0)),
            pl.BlockSpec((_BN, _K), lambda n: (n, 0)),
            pl.BlockSpec((_BN * _Q, _D), lambda n: (n, 0)),
        ],
        out_specs=[
            pl.BlockSpec((1, _B), lambda n: (0, 0)),
            pl.BlockSpec((_BN * _Q, _D), lambda n: (n, 0)),
        ],
        out_shape=[
            jax.ShapeDtypeStruct((1, _B), jnp.float32),
            jax.ShapeDtypeStruct((_N * _Q, _D), jnp.float32),
        ],
        scratch_shapes=[
            pltpu.VMEM((1, _B), jnp.float32),
        ],
    )(qt, g_flat, ctx2d)

    return (out1.reshape(_B), ctx_out)
